# baseline (speedup 1.0000x reference)
import functools
import math

import jax
import jax.numpy as jnp
from jax import lax
from jax.experimental import pallas as pl
from jax.experimental.pallas import tpu as pltpu
from jax.experimental.pallas import tpu_sc as plsc

N_RAYS = 4096
MAX_STEP = 1024
GRID = 128
NCELL = GRID * GRID * GRID
CONE = 0.00390625
NEAR = 0.05
MIN_OPT_THICKNESS = 0.01
DT_MIN = math.sqrt(3.0) / 1024.0
DT_MAX = math.sqrt(3.0) / 16.0

RB_A = 256
RB_C = 256
CCHUNK = 1024
K_SC = 8192


def _ray_setup(oxv, oyv, ozv, dxv, dyv, dzv):
    norm = jnp.sqrt(dxv * dxv + dyv * dyv + dzv * dzv) + 1e-9
    ux, uy, uz = dxv / norm, dyv / norm, dzv / norm

    def inv(u):
        return 1.0 / jnp.where(jnp.abs(u) > 1e-9, u, 1e-9)

    ix, iy, iz = inv(ux), inv(uy), inv(uz)
    t0x, t1x = (0.0 - oxv) * ix, (1.0 - oxv) * ix
    t0y, t1y = (0.0 - oyv) * iy, (1.0 - oyv) * iy
    t0z, t1z = (0.0 - ozv) * iz, (1.0 - ozv) * iz
    tmin = jnp.maximum(
        jnp.maximum(jnp.minimum(t0x, t1x), jnp.minimum(t0y, t1y)),
        jnp.minimum(t0z, t1z))
    tmin = jnp.maximum(tmin, NEAR)
    tmax = jnp.minimum(
        jnp.minimum(jnp.maximum(t0x, t1x), jnp.maximum(t0y, t1y)),
        jnp.maximum(t0z, t1z))
    return ux, uy, uz, tmin, tmax


HALF = NCELL // 2


def _index_body(ox, oy, oz, dx, dy, dz, g, s, flat_ref, idx_ref):
    ux, uy, uz, tmin, _ = _ray_setup(
        ox[...], oy[...], oz[...], dx[...], dy[...], dz[...])
    t = jnp.maximum(tmin * g[...], tmin + s[...])
    px = ox[...] + ux * t
    py = oy[...] + uy * t
    pz = oz[...] + uz * t

    def cell(p):
        return jnp.clip((p * float(GRID)).astype(jnp.int32), 0, GRID - 1)

    flat = (cell(px) * GRID + cell(py)) * GRID + cell(pz)
    flat_ref[...] = flat
    idx_ref[0, :, :] = jnp.minimum(flat, HALF - 1)
    idx_ref[1, :, :] = jnp.maximum(flat - HALF, 0)


def _index_call(ox, oy, oz, dx, dy, dz, g, s):
    nb = N_RAYS // RB_A
    ray_spec = pl.BlockSpec((RB_A, 1), lambda i: (i, 0))
    const_spec = pl.BlockSpec((1, MAX_STEP), lambda i: (0, 0))
    return pl.pallas_call(
        _index_body,
        grid=(nb,),
        in_specs=[ray_spec] * 6 + [const_spec] * 2,
        out_specs=[
            pl.BlockSpec((RB_A, MAX_STEP), lambda i: (i, 0)),
            pl.BlockSpec((2, RB_A, MAX_STEP), lambda i: (0, i, 0)),
        ],
        out_shape=[
            jax.ShapeDtypeStruct((N_RAYS, MAX_STEP), jnp.int32),
            jax.ShapeDtypeStruct((2, N_RAYS, MAX_STEP), jnp.int32),
        ],
    )(ox, oy, oz, dx, dy, dz, g, s)


def _gridsum_body(x_ref, acc_ref):
    @pl.when(pl.program_id(0) == 0)
    def _():
        acc_ref[...] = jnp.zeros_like(acc_ref)

    acc_ref[...] += jnp.sum(x_ref[...], axis=(0, 1), keepdims=True)


def _gridsum_call(grid2d):
    nb = grid2d.shape[0] // 128
    return pl.pallas_call(
        _gridsum_body,
        grid=(nb,),
        in_specs=[pl.BlockSpec((128, grid2d.shape[1]), lambda i: (i, 0))],
        out_specs=pl.BlockSpec((1, 1), lambda i: (0, 0)),
        out_shape=jax.ShapeDtypeStruct((1, 1), jnp.float32),
    )(grid2d)


def _sc_gather_call(idx2, density_grid):
    n = idx2.shape[1]
    info = plsc.get_sparse_core_info()
    nc, ns = info.num_cores, info.num_subcores
    per_s = n // ns
    chunks = per_s // K_SC
    mesh = plsc.VectorSubcoreMesh(core_axis_name="c", subcore_axis_name="s")

    assert chunks >= 2 and chunks % 2 == 0

    @functools.partial(
        pl.kernel,
        mesh=mesh,
        out_type=jax.ShapeDtypeStruct((nc, n), jnp.float32),
        scratch_types=[
            pltpu.VMEM((K_SC,), jnp.int32),
            pltpu.VMEM((K_SC,), jnp.int32),
            pltpu.VMEM((K_SC,), jnp.float32),
            pltpu.VMEM((K_SC,), jnp.float32),
            pltpu.VMEM_SHARED((HALF,), jnp.float32),
            pltpu.SemaphoreType.DMA,
            pltpu.SemaphoreType.DMA,
            pltpu.SemaphoreType.DMA,
            pltpu.SemaphoreType.DMA,
            pltpu.SemaphoreType.DMA,
            pltpu.SemaphoreType.DMA,
        ],
    )
    def k(idx_hbm, grid_hbm, out_hbm,
          idx0, idx1, dens0, dens1, spm, si0, si1, sg0, sg1, so0, so1):
        core = lax.axis_index("c")
        sub = lax.axis_index("s")
        base = sub * per_s
        idx_v = (idx0, idx1)
        dens_v = (dens0, dens1)
        sem_i = (si0, si1)
        sem_g = (sg0, sg1)
        sem_o = (so0, so1)

        @pl.when(sub == 0)
        def _():
            pltpu.sync_copy(grid_hbm.at[pl.ds(core * HALF, HALF)], spm)

        plsc.subcore_barrier()

        def idx_copy(c, b):
            return pltpu.make_async_copy(
                idx_hbm.at[core, pl.ds(base + c * K_SC, K_SC)],
                idx_v[b], sem_i[b])

        def out_copy(c, b):
            return pltpu.make_async_copy(
                dens_v[b], out_hbm.at[core, pl.ds(base + c * K_SC, K_SC)],
                sem_o[b])

        idx_copy(0, 0).start()

        def body(g, carry):
            for b in range(2):
                c = g * 2 + b
                @pl.when(g >= 1)
                def _():
                    out_copy(c - 2, b).wait()

                idx_copy(c, b).wait()
                if b == 0:
                    idx_copy(c + 1, 1).start()
                else:
                    @pl.when(g < chunks // 2 - 1)
                    def _():
                        idx_copy(c + 1, 0).start()

                gather = pltpu.make_async_copy(
                    spm.at[idx_v[b]], dens_v[b], sem_g[b])
                gather.start()
                gather.wait()
                out_copy(c, b).start()
            return carry

        lax.fori_loop(0, chunks // 2, body, 0)
        out_copy(chunks - 2, 0).wait()
        out_copy(chunks - 1, 1).wait()

    return k(idx2, density_grid)


def _assemble_body(ox, oy, oz, dx, dy, dz, flatb, densa, densb, gsum, gi, si,
                   out_ref, nst_ref):
    c = pl.program_id(1)
    ux, uy, uz, tmin, tmax = _ray_setup(
        ox[...], oy[...], oz[...], dx[...], dy[...], dz[...])
    dens = jnp.where(flatb[...] < HALF, densa[0], densb[0])
    t = jnp.maximum(tmin * gi[...], tmin + si[...])
    dt = jnp.clip(t * CONE, DT_MIN, DT_MAX)
    px = ox[...] + ux * t
    py = oy[...] + uy * t
    pz = oz[...] + uz * t
    valid = ((t < tmax)
             & (px >= 0.0) & (px < 1.0)
             & (py >= 0.0) & (py < 1.0)
             & (pz >= 0.0) & (pz < 1.0))

    rowi = lax.broadcasted_iota(jnp.int32, (128, CCHUNK), 0)
    coli = lax.broadcasted_iota(jnp.int32, (128, CCHUNK), 1)
    expand = (rowi == coli // 8).astype(jnp.float32)
    densi = lax.dot_general(
        dens, expand, (((1,), (0,)), ((), ())),
        preferred_element_type=jnp.float32,
        precision=lax.Precision.HIGHEST)

    mean = gsum[0, 0] / float(NCELL)
    thresh = jnp.minimum(mean, MIN_OPT_THICKNESS)
    occ = valid & (densi > thresh)
    occf = occ.astype(jnp.float32)

    chan = lax.broadcasted_iota(jnp.int32, (RB_C, CCHUNK), 1) % 8
    ones = jnp.ones_like(t)
    res = jnp.where(
        chan == 0, px,
        jnp.where(chan == 1, py,
                  jnp.where(chan == 2, pz,
                            jnp.where(chan == 3, dt,
                                      jnp.where(chan == 4, ux * ones,
                                                jnp.where(chan == 5, uy * ones,
                                                          jnp.where(chan == 6, uz * ones,
                                                                    densi)))))))
    out_ref[...] = res * occf

    @pl.when(c == 0)
    def _():
        nst_ref[...] = jnp.zeros_like(nst_ref)

    nst_ref[...] += jnp.sum(occ.astype(jnp.int32), axis=1, keepdims=True) // 8


def _assemble_call(ox, oy, oz, dx, dy, dz, flat, dens2, gsum, gi, si):
    nb = N_RAYS // RB_C
    ncc = (MAX_STEP * 8) // CCHUNK
    ray_spec = pl.BlockSpec((RB_C, 1), lambda i, c: (i, 0))
    step_chunk = CCHUNK // 8
    out2, nst = pl.pallas_call(
        _assemble_body,
        grid=(nb, ncc),
        in_specs=[ray_spec] * 6 + [
            pl.BlockSpec((RB_C, step_chunk), lambda i, c: (i, c)),
            pl.BlockSpec((1, RB_C, step_chunk), lambda i, c: (0, i, c)),
            pl.BlockSpec((1, RB_C, step_chunk), lambda i, c: (1, i, c)),
            pl.BlockSpec((1, 1), lambda i, c: (0, 0)),
            pl.BlockSpec((1, CCHUNK), lambda i, c: (0, c)),
            pl.BlockSpec((1, CCHUNK), lambda i, c: (0, c)),
        ],
        out_specs=[
            pl.BlockSpec((RB_C, CCHUNK), lambda i, c: (i, c)),
            pl.BlockSpec((RB_C, 1), lambda i, c: (i, 0)),
        ],
        out_shape=[
            jax.ShapeDtypeStruct((N_RAYS, MAX_STEP * 8), jnp.float32),
            jax.ShapeDtypeStruct((N_RAYS, 1), jnp.int32),
        ],
    )(ox, oy, oz, dx, dy, dz, flat, dens2, dens2, gsum, gi, si)
    return out2, nst


def kernel(rays_o, rays_d, density_grid):
    ox, oy, oz = (rays_o[:, i:i + 1] for i in range(3))
    dx, dy, dz = (rays_d[:, i:i + 1] for i in range(3))
    steps = jnp.arange(MAX_STEP, dtype=jnp.float32)
    g = jnp.power(1.0 + CONE, steps).reshape(1, MAX_STEP)
    s = (steps * DT_MIN).reshape(1, MAX_STEP)
    gi = jnp.repeat(g, 8, axis=1)
    si = jnp.repeat(s, 8, axis=1)

    flat, idx2 = _index_call(ox, oy, oz, dx, dy, dz, g, s)
    gsum = _gridsum_call(density_grid.reshape(NCELL // MAX_STEP, MAX_STEP))
    dens2 = _sc_gather_call(
        idx2.reshape(2, N_RAYS * MAX_STEP), density_grid)
    out2, nst = _assemble_call(
        ox, oy, oz, dx, dy, dz, flat, dens2.reshape(2, N_RAYS, MAX_STEP),
        gsum, gi, si)
    coords = out2.reshape(N_RAYS, MAX_STEP, 8)
    return coords, nst.reshape(N_RAYS)


# speedup vs baseline: 1.2319x; 1.2319x over previous
import functools
import math

import jax
import jax.numpy as jnp
from jax import lax
from jax.experimental import pallas as pl
from jax.experimental.pallas import tpu as pltpu
from jax.experimental.pallas import tpu_sc as plsc

N_RAYS = 4096
MAX_STEP = 1024
GRID = 128
NCELL = GRID * GRID * GRID
CONE = 0.00390625
NEAR = 0.05
MIN_OPT_THICKNESS = 0.01
DT_MIN = math.sqrt(3.0) / 1024.0
DT_MAX = math.sqrt(3.0) / 16.0

N_SLAB = 4
SLAB = N_RAYS // N_SLAB
RB_A = 256
RB_C = 256
CCHUNK = 1024
K_SC = 8192


def _ray_setup(oxv, oyv, ozv, dxv, dyv, dzv):
    norm = jnp.sqrt(dxv * dxv + dyv * dyv + dzv * dzv) + 1e-9
    ux, uy, uz = dxv / norm, dyv / norm, dzv / norm

    def inv(u):
        return 1.0 / jnp.where(jnp.abs(u) > 1e-9, u, 1e-9)

    ix, iy, iz = inv(ux), inv(uy), inv(uz)
    t0x, t1x = (0.0 - oxv) * ix, (1.0 - oxv) * ix
    t0y, t1y = (0.0 - oyv) * iy, (1.0 - oyv) * iy
    t0z, t1z = (0.0 - ozv) * iz, (1.0 - ozv) * iz
    tmin = jnp.maximum(
        jnp.maximum(jnp.minimum(t0x, t1x), jnp.minimum(t0y, t1y)),
        jnp.minimum(t0z, t1z))
    tmin = jnp.maximum(tmin, NEAR)
    tmax = jnp.minimum(
        jnp.minimum(jnp.maximum(t0x, t1x), jnp.maximum(t0y, t1y)),
        jnp.maximum(t0z, t1z))
    return ux, uy, uz, tmin, tmax


def _index_body(ox, oy, oz, dx, dy, dz, g, s, flat_ref):
    ux, uy, uz, tmin, _ = _ray_setup(
        ox[...], oy[...], oz[...], dx[...], dy[...], dz[...])
    t = jnp.maximum(tmin * g[...], tmin + s[...])
    px = ox[...] + ux * t
    py = oy[...] + uy * t
    pz = oz[...] + uz * t

    def cell(p):
        return jnp.clip((p * float(GRID)).astype(jnp.int32), 0, GRID - 1)

    flat_ref[...] = (cell(px) * GRID + cell(py)) * GRID + cell(pz)


def _index_call(ox, oy, oz, dx, dy, dz, g, s):
    nb = SLAB // RB_A
    ray_spec = pl.BlockSpec((RB_A, 1), lambda i: (i, 0))
    const_spec = pl.BlockSpec((1, MAX_STEP), lambda i: (0, 0))
    return pl.pallas_call(
        _index_body,
        grid=(nb,),
        in_specs=[ray_spec] * 6 + [const_spec] * 2,
        out_specs=pl.BlockSpec((RB_A, MAX_STEP), lambda i: (i, 0)),
        out_shape=jax.ShapeDtypeStruct((SLAB, MAX_STEP), jnp.int32),
    )(ox, oy, oz, dx, dy, dz, g, s)


def _gridsum_body(x_ref, acc_ref):
    @pl.when(pl.program_id(0) == 0)
    def _():
        acc_ref[...] = jnp.zeros_like(acc_ref)

    acc_ref[...] += jnp.sum(x_ref[...], axis=(0, 1), keepdims=True)


def _gridsum_call(grid2d):
    nb = grid2d.shape[0] // 128
    return pl.pallas_call(
        _gridsum_body,
        grid=(nb,),
        in_specs=[pl.BlockSpec((128, grid2d.shape[1]), lambda i: (i, 0))],
        out_specs=pl.BlockSpec((1, 1), lambda i: (0, 0)),
        out_shape=jax.ShapeDtypeStruct((1, 1), jnp.float32),
    )(grid2d)


def _sc_gather_call(flat_idx, density_grid):
    n = flat_idx.shape[0]
    info = plsc.get_sparse_core_info()
    nc, ns = info.num_cores, info.num_subcores
    nw = nc * ns
    per_w = n // nw
    chunks = per_w // K_SC
    mesh = plsc.VectorSubcoreMesh(core_axis_name="c", subcore_axis_name="s")

    assert chunks >= 2 and chunks % 2 == 0

    @functools.partial(
        pl.kernel,
        mesh=mesh,
        out_type=jax.ShapeDtypeStruct((n,), jnp.float32),
        scratch_types=[
            pltpu.VMEM((K_SC,), jnp.int32),
            pltpu.VMEM((K_SC,), jnp.int32),
            pltpu.VMEM((K_SC,), jnp.float32),
            pltpu.VMEM((K_SC,), jnp.float32),
            pltpu.SemaphoreType.DMA,
            pltpu.SemaphoreType.DMA,
            pltpu.SemaphoreType.DMA,
            pltpu.SemaphoreType.DMA,
            pltpu.SemaphoreType.DMA,
            pltpu.SemaphoreType.DMA,
        ],
    )
    def k(idx_hbm, grid_hbm, out_hbm,
          idx0, idx1, dens0, dens1, si0, si1, sg0, sg1, so0, so1):
        wid = lax.axis_index("s") * nc + lax.axis_index("c")
        base = wid * per_w
        idx_v = (idx0, idx1)
        dens_v = (dens0, dens1)
        sem_i = (si0, si1)
        sem_g = (sg0, sg1)
        sem_o = (so0, so1)

        def idx_copy(c, b):
            return pltpu.make_async_copy(
                idx_hbm.at[pl.ds(base + c * K_SC, K_SC)], idx_v[b], sem_i[b])

        def out_copy(c, b):
            return pltpu.make_async_copy(
                dens_v[b], out_hbm.at[pl.ds(base + c * K_SC, K_SC)], sem_o[b])

        idx_copy(0, 0).start()

        def body(g, carry):
            for b in range(2):
                c = g * 2 + b
                @pl.when(g >= 1)
                def _():
                    out_copy(c - 2, b).wait()

                idx_copy(c, b).wait()
                if b == 0:
                    idx_copy(c + 1, 1).start()
                else:
                    @pl.when(g < chunks // 2 - 1)
                    def _():
                        idx_copy(c + 1, 0).start()

                gather = pltpu.make_async_copy(
                    grid_hbm.at[idx_v[b]], dens_v[b], sem_g[b])
                gather.start()
                gather.wait()
                out_copy(c, b).start()
            return carry

        lax.fori_loop(0, chunks // 2, body, 0)
        out_copy(chunks - 2, 0).wait()
        out_copy(chunks - 1, 1).wait()

    return k(flat_idx, density_grid)


def _assemble_body(ox, oy, oz, dx, dy, dz, dens, gsum, gi, si, o2a, nsa,
                   out_ref, nst_ref):
    c = pl.program_id(1)
    ux, uy, uz, tmin, tmax = _ray_setup(
        ox[...], oy[...], oz[...], dx[...], dy[...], dz[...])
    t = jnp.maximum(tmin * gi[...], tmin + si[...])
    dt = jnp.clip(t * CONE, DT_MIN, DT_MAX)
    px = ox[...] + ux * t
    py = oy[...] + uy * t
    pz = oz[...] + uz * t
    valid = ((t < tmax)
             & (px >= 0.0) & (px < 1.0)
             & (py >= 0.0) & (py < 1.0)
             & (pz >= 0.0) & (pz < 1.0))

    rowi = lax.broadcasted_iota(jnp.int32, (128, CCHUNK), 0)
    coli = lax.broadcasted_iota(jnp.int32, (128, CCHUNK), 1)
    expand = (rowi == coli // 8).astype(jnp.float32)
    densi = lax.dot_general(
        dens[...], expand, (((1,), (0,)), ((), ())),
        preferred_element_type=jnp.float32,
        precision=lax.Precision.HIGHEST)

    mean = gsum[0, 0] / float(NCELL)
    thresh = jnp.minimum(mean, MIN_OPT_THICKNESS)
    occ = valid & (densi > thresh)
    occf = occ.astype(jnp.float32)

    chan = lax.broadcasted_iota(jnp.int32, (RB_C, CCHUNK), 1) % 8
    ones = jnp.ones_like(t)
    res = jnp.where(
        chan == 0, px,
        jnp.where(chan == 1, py,
                  jnp.where(chan == 2, pz,
                            jnp.where(chan == 3, dt,
                                      jnp.where(chan == 4, ux * ones,
                                                jnp.where(chan == 5, uy * ones,
                                                          jnp.where(chan == 6, uz * ones,
                                                                    densi)))))))
    out_ref[...] = res * occf

    @pl.when(c == 0)
    def _():
        nst_ref[...] = jnp.zeros_like(nst_ref)

    nst_ref[...] += jnp.sum(occ.astype(jnp.int32), axis=1, keepdims=True) // 8


def _assemble_call(slab, ox, oy, oz, dx, dy, dz, dens, gsum, gi, si,
                   out2_acc, nst_acc):
    nb = SLAB // RB_C
    ncc = (MAX_STEP * 8) // CCHUNK
    sb = slab * nb
    ray_spec = pl.BlockSpec((RB_C, 1), lambda i, c: (i, 0))
    any_spec = pl.BlockSpec(memory_space=pl.ANY)
    out2, nst = pl.pallas_call(
        _assemble_body,
        grid=(nb, ncc),
        in_specs=[ray_spec] * 6 + [
            pl.BlockSpec((RB_C, CCHUNK // 8), lambda i, c: (i, c)),
            pl.BlockSpec((1, 1), lambda i, c: (0, 0)),
            pl.BlockSpec((1, CCHUNK), lambda i, c: (0, c)),
            pl.BlockSpec((1, CCHUNK), lambda i, c: (0, c)),
            any_spec,
            any_spec,
        ],
        out_specs=[
            pl.BlockSpec((RB_C, CCHUNK), lambda i, c: (i + sb, c)),
            pl.BlockSpec((RB_C, 1), lambda i, c: (i + sb, 0)),
        ],
        out_shape=[
            jax.ShapeDtypeStruct((N_RAYS, MAX_STEP * 8), jnp.float32),
            jax.ShapeDtypeStruct((N_RAYS, 1), jnp.int32),
        ],
        input_output_aliases={10: 0, 11: 1},
    )(ox, oy, oz, dx, dy, dz, dens, gsum, gi, si, out2_acc, nst_acc)
    return out2, nst


def kernel(rays_o, rays_d, density_grid):
    ox, oy, oz = (rays_o[:, i:i + 1] for i in range(3))
    dx, dy, dz = (rays_d[:, i:i + 1] for i in range(3))
    steps = jnp.arange(MAX_STEP, dtype=jnp.float32)
    g = jnp.power(1.0 + CONE, steps).reshape(1, MAX_STEP)
    s = (steps * DT_MIN).reshape(1, MAX_STEP)
    gi = jnp.repeat(g, 8, axis=1)
    si = jnp.repeat(s, 8, axis=1)

    gsum = _gridsum_call(density_grid.reshape(NCELL // MAX_STEP, MAX_STEP))

    def sl(a, k):
        return a[k * SLAB:(k + 1) * SLAB]

    dens_slabs = []
    rays_slabs = []
    for k in range(N_SLAB):
        rs = tuple(sl(a, k) for a in (ox, oy, oz, dx, dy, dz))
        flat = _index_call(*rs, g, s)
        dens_slabs.append(_sc_gather_call(flat.reshape(-1), density_grid))
        rays_slabs.append(rs)

    out2 = jnp.zeros((N_RAYS, MAX_STEP * 8), jnp.float32)
    nst = jnp.zeros((N_RAYS, 1), jnp.int32)
    for k in range(N_SLAB):
        out2, nst = _assemble_call(
            k, *rays_slabs[k], dens_slabs[k].reshape(SLAB, MAX_STEP),
            gsum, gi, si, out2, nst)

    coords = out2.reshape(N_RAYS, MAX_STEP, 8)
    return coords, nst.reshape(N_RAYS)


# speedup vs baseline: 1.2873x; 1.0449x over previous
import functools
import math

import jax
import jax.numpy as jnp
from jax import lax
from jax.experimental import pallas as pl
from jax.experimental.pallas import tpu as pltpu
from jax.experimental.pallas import tpu_sc as plsc

N_RAYS = 4096
MAX_STEP = 1024
GRID = 128
NCELL = GRID * GRID * GRID
CONE = 0.00390625
NEAR = 0.05
MIN_OPT_THICKNESS = 0.01
DT_MIN = math.sqrt(3.0) / 1024.0
DT_MAX = math.sqrt(3.0) / 16.0

SLABS = (1280, 1280, 1280, 256)
RB_A = 256
RB_C = 256
CCHUNK = 1024
K_SC = 4096


def _ray_setup(oxv, oyv, ozv, dxv, dyv, dzv):
    norm = jnp.sqrt(dxv * dxv + dyv * dyv + dzv * dzv) + 1e-9
    ux, uy, uz = dxv / norm, dyv / norm, dzv / norm

    def inv(u):
        return 1.0 / jnp.where(jnp.abs(u) > 1e-9, u, 1e-9)

    ix, iy, iz = inv(ux), inv(uy), inv(uz)
    t0x, t1x = (0.0 - oxv) * ix, (1.0 - oxv) * ix
    t0y, t1y = (0.0 - oyv) * iy, (1.0 - oyv) * iy
    t0z, t1z = (0.0 - ozv) * iz, (1.0 - ozv) * iz
    tmin = jnp.maximum(
        jnp.maximum(jnp.minimum(t0x, t1x), jnp.minimum(t0y, t1y)),
        jnp.minimum(t0z, t1z))
    tmin = jnp.maximum(tmin, NEAR)
    tmax = jnp.minimum(
        jnp.minimum(jnp.maximum(t0x, t1x), jnp.maximum(t0y, t1y)),
        jnp.maximum(t0z, t1z))
    return ux, uy, uz, tmin, tmax


def _index_body(ox, oy, oz, dx, dy, dz, g, s, flat_ref):
    ux, uy, uz, tmin, _ = _ray_setup(
        ox[...], oy[...], oz[...], dx[...], dy[...], dz[...])
    t = jnp.maximum(tmin * g[...], tmin + s[...])
    px = ox[...] + ux * t
    py = oy[...] + uy * t
    pz = oz[...] + uz * t

    def cell(p):
        return jnp.clip((p * float(GRID)).astype(jnp.int32), 0, GRID - 1)

    flat_ref[...] = (cell(px) * GRID + cell(py)) * GRID + cell(pz)


def _index_call(ox, oy, oz, dx, dy, dz, g, s):
    slab = ox.shape[0]
    nb = slab // RB_A
    ray_spec = pl.BlockSpec((RB_A, 1), lambda i: (i, 0))
    const_spec = pl.BlockSpec((1, MAX_STEP), lambda i: (0, 0))
    return pl.pallas_call(
        _index_body,
        grid=(nb,),
        in_specs=[ray_spec] * 6 + [const_spec] * 2,
        out_specs=pl.BlockSpec((RB_A, MAX_STEP), lambda i: (i, 0)),
        out_shape=jax.ShapeDtypeStruct((slab, MAX_STEP), jnp.int32),
    )(ox, oy, oz, dx, dy, dz, g, s)


def _gridsum_body(x_ref, acc_ref):
    @pl.when(pl.program_id(0) == 0)
    def _():
        acc_ref[...] = jnp.zeros_like(acc_ref)

    acc_ref[...] += jnp.sum(x_ref[...], axis=(0, 1), keepdims=True)


def _gridsum_call(grid2d):
    nb = grid2d.shape[0] // 128
    return pl.pallas_call(
        _gridsum_body,
        grid=(nb,),
        in_specs=[pl.BlockSpec((128, grid2d.shape[1]), lambda i: (i, 0))],
        out_specs=pl.BlockSpec((1, 1), lambda i: (0, 0)),
        out_shape=jax.ShapeDtypeStruct((1, 1), jnp.float32),
    )(grid2d)


def _sc_gather_call(flat_idx, density_grid):
    n = flat_idx.shape[0]
    info = plsc.get_sparse_core_info()
    nc, ns = info.num_cores, info.num_subcores
    nw = nc * ns
    per_w = n // nw
    chunks = per_w // K_SC
    mesh = plsc.VectorSubcoreMesh(core_axis_name="c", subcore_axis_name="s")

    assert chunks >= 2 and chunks % 2 == 0

    @functools.partial(
        pl.kernel,
        mesh=mesh,
        out_type=jax.ShapeDtypeStruct((n,), jnp.float32),
        scratch_types=[
            pltpu.VMEM((K_SC,), jnp.int32),
            pltpu.VMEM((K_SC,), jnp.int32),
            pltpu.VMEM((K_SC,), jnp.float32),
            pltpu.VMEM((K_SC,), jnp.float32),
            pltpu.SemaphoreType.DMA,
            pltpu.SemaphoreType.DMA,
            pltpu.SemaphoreType.DMA,
            pltpu.SemaphoreType.DMA,
            pltpu.SemaphoreType.DMA,
            pltpu.SemaphoreType.DMA,
        ],
    )
    def k(idx_hbm, grid_hbm, out_hbm,
          idx0, idx1, dens0, dens1, si0, si1, sg0, sg1, so0, so1):
        wid = lax.axis_index("s") * nc + lax.axis_index("c")
        base = wid * per_w
        idx_v = (idx0, idx1)
        dens_v = (dens0, dens1)
        sem_i = (si0, si1)
        sem_g = (sg0, sg1)
        sem_o = (so0, so1)

        def idx_copy(c, b):
            return pltpu.make_async_copy(
                idx_hbm.at[pl.ds(base + c * K_SC, K_SC)], idx_v[b], sem_i[b])

        def out_copy(c, b):
            return pltpu.make_async_copy(
                dens_v[b], out_hbm.at[pl.ds(base + c * K_SC, K_SC)], sem_o[b])

        idx_copy(0, 0).start()

        def body(g, carry):
            for b in range(2):
                c = g * 2 + b
                @pl.when(g >= 1)
                def _():
                    out_copy(c - 2, b).wait()

                idx_copy(c, b).wait()
                if b == 0:
                    idx_copy(c + 1, 1).start()
                else:
                    @pl.when(g < chunks // 2 - 1)
                    def _():
                        idx_copy(c + 1, 0).start()

                gather = pltpu.make_async_copy(
                    grid_hbm.at[idx_v[b]], dens_v[b], sem_g[b])
                gather.start()
                gather.wait()
                out_copy(c, b).start()
            return carry

        lax.fori_loop(0, chunks // 2, body, 0)
        out_copy(chunks - 2, 0).wait()
        out_copy(chunks - 1, 1).wait()

    return k(flat_idx, density_grid)


def _assemble_body(ox, oy, oz, dx, dy, dz, dens, gsum, gi, si, o2a, nsa,
                   out_ref, nst_ref):
    c = pl.program_id(1)
    ux, uy, uz, tmin, tmax = _ray_setup(
        ox[...], oy[...], oz[...], dx[...], dy[...], dz[...])
    t = jnp.maximum(tmin * gi[...], tmin + si[...])
    dt = jnp.clip(t * CONE, DT_MIN, DT_MAX)
    px = ox[...] + ux * t
    py = oy[...] + uy * t
    pz = oz[...] + uz * t
    valid = ((t < tmax)
             & (px >= 0.0) & (px < 1.0)
             & (py >= 0.0) & (py < 1.0)
             & (pz >= 0.0) & (pz < 1.0))

    rowi = lax.broadcasted_iota(jnp.int32, (128, CCHUNK), 0)
    coli = lax.broadcasted_iota(jnp.int32, (128, CCHUNK), 1)
    expand = (rowi == coli // 8).astype(jnp.float32)
    densi = lax.dot_general(
        dens[...], expand, (((1,), (0,)), ((), ())),
        preferred_element_type=jnp.float32,
        precision=lax.Precision.HIGHEST)

    mean = gsum[0, 0] / float(NCELL)
    thresh = jnp.minimum(mean, MIN_OPT_THICKNESS)
    occ = valid & (densi > thresh)
    occf = occ.astype(jnp.float32)

    chan = lax.broadcasted_iota(jnp.int32, (RB_C, CCHUNK), 1) % 8
    ones = jnp.ones_like(t)
    res = jnp.where(
        chan == 0, px,
        jnp.where(chan == 1, py,
                  jnp.where(chan == 2, pz,
                            jnp.where(chan == 3, dt,
                                      jnp.where(chan == 4, ux * ones,
                                                jnp.where(chan == 5, uy * ones,
                                                          jnp.where(chan == 6, uz * ones,
                                                                    densi)))))))
    out_ref[...] = res * occf

    @pl.when(c == 0)
    def _():
        nst_ref[...] = jnp.zeros_like(nst_ref)

    nst_ref[...] += jnp.sum(occ.astype(jnp.int32), axis=1, keepdims=True) // 8


def _assemble_call(ray_start, ox, oy, oz, dx, dy, dz, dens, gsum, gi, si,
                   out2_acc=None, nst_acc=None):
    slab = ox.shape[0]
    nb = slab // RB_C
    ncc = (MAX_STEP * 8) // CCHUNK
    sb = ray_start // RB_C
    ray_spec = pl.BlockSpec((RB_C, 1), lambda i, c: (i, 0))
    any_spec = pl.BlockSpec(memory_space=pl.ANY)
    in_specs = [ray_spec] * 6 + [
        pl.BlockSpec((RB_C, CCHUNK // 8), lambda i, c: (i, c)),
        pl.BlockSpec((1, 1), lambda i, c: (0, 0)),
        pl.BlockSpec((1, CCHUNK), lambda i, c: (0, c)),
        pl.BlockSpec((1, CCHUNK), lambda i, c: (0, c)),
    ]
    args = [ox, oy, oz, dx, dy, dz, dens, gsum, gi, si]
    aliases = {}
    if out2_acc is not None:
        in_specs += [any_spec, any_spec]
        args += [out2_acc, nst_acc]
        aliases = {10: 0, 11: 1}
    body = _assemble_body if out2_acc is not None else (
        lambda *a: _assemble_body(*a[:10], None, None, *a[10:]))
    out2, nst = pl.pallas_call(
        body,
        grid=(nb, ncc),
        in_specs=in_specs,
        out_specs=[
            pl.BlockSpec((RB_C, CCHUNK), lambda i, c: (i + sb, c)),
            pl.BlockSpec((RB_C, 1), lambda i, c: (i + sb, 0)),
        ],
        out_shape=[
            jax.ShapeDtypeStruct((N_RAYS, MAX_STEP * 8), jnp.float32),
            jax.ShapeDtypeStruct((N_RAYS, 1), jnp.int32),
        ],
        input_output_aliases=aliases,
    )(*args)
    return out2, nst


def kernel(rays_o, rays_d, density_grid):
    ox, oy, oz = (rays_o[:, i:i + 1] for i in range(3))
    dx, dy, dz = (rays_d[:, i:i + 1] for i in range(3))
    steps = jnp.arange(MAX_STEP, dtype=jnp.float32)
    g = jnp.power(1.0 + CONE, steps).reshape(1, MAX_STEP)
    s = (steps * DT_MIN).reshape(1, MAX_STEP)
    gi = jnp.repeat(g, 8, axis=1)
    si = jnp.repeat(s, 8, axis=1)

    gsum = _gridsum_call(density_grid.reshape(NCELL // MAX_STEP, MAX_STEP))

    starts = [sum(SLABS[:k]) for k in range(len(SLABS))]
    dens_slabs = []
    rays_slabs = []
    for k, (st, sz) in enumerate(zip(starts, SLABS)):
        rs = tuple(a[st:st + sz] for a in (ox, oy, oz, dx, dy, dz))
        flat = _index_call(*rs, g, s)
        dens_slabs.append(_sc_gather_call(flat.reshape(-1), density_grid))
        rays_slabs.append(rs)

    out2, nst = None, None
    for k, (st, sz) in enumerate(zip(starts, SLABS)):
        out2, nst = _assemble_call(
            st, *rays_slabs[k], dens_slabs[k].reshape(sz, MAX_STEP),
            gsum, gi, si, out2, nst)

    coords = out2.reshape(N_RAYS, MAX_STEP, 8)
    return coords, nst.reshape(N_RAYS)


# speedup vs baseline: 1.2947x; 1.0057x over previous
import functools
import math

import jax
import jax.numpy as jnp
from jax import lax
from jax.experimental import pallas as pl
from jax.experimental.pallas import tpu as pltpu
from jax.experimental.pallas import tpu_sc as plsc

N_RAYS = 4096
MAX_STEP = 1024
GRID = 128
NCELL = GRID * GRID * GRID
CONE = 0.00390625
NEAR = 0.05
MIN_OPT_THICKNESS = 0.01
DT_MIN = math.sqrt(3.0) / 1024.0
DT_MAX = math.sqrt(3.0) / 16.0

SLABS = (1280, 1280, 1280, 256)
RB_A = 256
RB_C = 256
CCHUNK = 1024
K_SC = 4096


def _ray_setup(oxv, oyv, ozv, dxv, dyv, dzv):
    norm = jnp.sqrt(dxv * dxv + dyv * dyv + dzv * dzv) + 1e-9
    ux, uy, uz = dxv / norm, dyv / norm, dzv / norm

    def inv(u):
        return 1.0 / jnp.where(jnp.abs(u) > 1e-9, u, 1e-9)

    ix, iy, iz = inv(ux), inv(uy), inv(uz)
    t0x, t1x = (0.0 - oxv) * ix, (1.0 - oxv) * ix
    t0y, t1y = (0.0 - oyv) * iy, (1.0 - oyv) * iy
    t0z, t1z = (0.0 - ozv) * iz, (1.0 - ozv) * iz
    tmin = jnp.maximum(
        jnp.maximum(jnp.minimum(t0x, t1x), jnp.minimum(t0y, t1y)),
        jnp.minimum(t0z, t1z))
    tmin = jnp.maximum(tmin, NEAR)
    tmax = jnp.minimum(
        jnp.minimum(jnp.maximum(t0x, t1x), jnp.maximum(t0y, t1y)),
        jnp.maximum(t0z, t1z))
    return ux, uy, uz, tmin, tmax


def _index_body(ox, oy, oz, dx, dy, dz, g, s, flat_ref):
    ux, uy, uz, tmin, _ = _ray_setup(
        ox[...], oy[...], oz[...], dx[...], dy[...], dz[...])
    t = jnp.maximum(tmin * g[...], tmin + s[...])
    px = ox[...] + ux * t
    py = oy[...] + uy * t
    pz = oz[...] + uz * t

    def cell(p):
        return jnp.clip((p * float(GRID)).astype(jnp.int32), 0, GRID - 1)

    flat_ref[...] = (cell(px) * GRID + cell(py)) * GRID + cell(pz)


def _index_call(ox, oy, oz, dx, dy, dz, g, s):
    slab = ox.shape[0]
    nb = slab // RB_A
    ray_spec = pl.BlockSpec((RB_A, 1), lambda i: (i, 0))
    const_spec = pl.BlockSpec((1, MAX_STEP), lambda i: (0, 0))
    return pl.pallas_call(
        _index_body,
        grid=(nb,),
        in_specs=[ray_spec] * 6 + [const_spec] * 2,
        out_specs=pl.BlockSpec((RB_A, MAX_STEP), lambda i: (i, 0)),
        out_shape=jax.ShapeDtypeStruct((slab, MAX_STEP), jnp.int32),
    )(ox, oy, oz, dx, dy, dz, g, s)


def _gridsum_body(x_ref, acc_ref):
    @pl.when(pl.program_id(0) == 0)
    def _():
        acc_ref[...] = jnp.zeros_like(acc_ref)

    acc_ref[...] += jnp.sum(x_ref[...], axis=(0, 1), keepdims=True)


def _gridsum_call(grid2d):
    nb = grid2d.shape[0] // 128
    return pl.pallas_call(
        _gridsum_body,
        grid=(nb,),
        in_specs=[pl.BlockSpec((128, grid2d.shape[1]), lambda i: (i, 0))],
        out_specs=pl.BlockSpec((1, 1), lambda i: (0, 0)),
        out_shape=jax.ShapeDtypeStruct((1, 1), jnp.float32),
    )(grid2d)


def _sc_gather_call(flat_idx, density_grid):
    n = flat_idx.shape[0]
    info = plsc.get_sparse_core_info()
    nc, ns = info.num_cores, info.num_subcores
    nw = nc * ns
    per_w = n // nw
    chunks = per_w // K_SC
    half = K_SC // 2
    mesh = plsc.VectorSubcoreMesh(core_axis_name="c", subcore_axis_name="s")

    assert chunks >= 2 and chunks % 2 == 0

    @functools.partial(
        pl.kernel,
        mesh=mesh,
        out_type=jax.ShapeDtypeStruct((n,), jnp.float32),
        scratch_types=[
            pltpu.VMEM((K_SC,), jnp.int32),
            pltpu.VMEM((K_SC,), jnp.int32),
            pltpu.VMEM((K_SC,), jnp.float32),
            pltpu.VMEM((K_SC,), jnp.float32),
            pltpu.SemaphoreType.DMA,
            pltpu.SemaphoreType.DMA,
            pltpu.SemaphoreType.DMA,
            pltpu.SemaphoreType.DMA,
            pltpu.SemaphoreType.DMA,
            pltpu.SemaphoreType.DMA,
            pltpu.SemaphoreType.DMA,
            pltpu.SemaphoreType.DMA,
        ],
    )
    def k(idx_hbm, grid_hbm, out_hbm,
          idx0, idx1, dens0, dens1,
          si0, si1, sg0, sg1, sh0, sh1, so0, so1):
        wid = lax.axis_index("s") * nc + lax.axis_index("c")
        base = wid * per_w
        idx_v = (idx0, idx1)
        dens_v = (dens0, dens1)
        sem_i = (si0, si1)
        sem_g = (sg0, sg1)
        sem_h = (sh0, sh1)
        sem_o = (so0, so1)

        def idx_copy(c, b):
            return pltpu.make_async_copy(
                idx_hbm.at[pl.ds(base + c * K_SC, K_SC)], idx_v[b], sem_i[b])

        def out_copy(c, b):
            return pltpu.make_async_copy(
                dens_v[b], out_hbm.at[pl.ds(base + c * K_SC, K_SC)], sem_o[b])

        def gather_a(b):
            return pltpu.make_async_copy(
                grid_hbm.at[idx_v[b].at[pl.ds(0, half)]],
                dens_v[b].at[pl.ds(0, half)], sem_g[b])

        def gather_b(b):
            return pltpu.make_async_copy(
                grid_hbm.at[idx_v[b].at[pl.ds(half, half)]],
                dens_v[b].at[pl.ds(half, half)], sem_h[b])

        idx_copy(0, 0).start()

        def body(g, carry):
            for b in range(2):
                c = g * 2 + b
                @pl.when(g >= 1)
                def _():
                    out_copy(c - 2, b).wait()

                idx_copy(c, b).wait()
                if b == 0:
                    idx_copy(c + 1, 1).start()
                else:
                    @pl.when(g < chunks // 2 - 1)
                    def _():
                        idx_copy(c + 1, 0).start()

                gather_a(b).start()
                gather_b(b).start()
                gather_a(b).wait()
                gather_b(b).wait()
                out_copy(c, b).start()
            return carry

        lax.fori_loop(0, chunks // 2, body, 0)
        out_copy(chunks - 2, 0).wait()
        out_copy(chunks - 1, 1).wait()

    return k(flat_idx, density_grid)


def _assemble_body(ox, oy, oz, dx, dy, dz, dens, gsum, gi, si, o2a, nsa,
                   out_ref, nst_ref):
    c = pl.program_id(1)
    ux, uy, uz, tmin, tmax = _ray_setup(
        ox[...], oy[...], oz[...], dx[...], dy[...], dz[...])
    t = jnp.maximum(tmin * gi[...], tmin + si[...])
    dt = jnp.clip(t * CONE, DT_MIN, DT_MAX)
    px = ox[...] + ux * t
    py = oy[...] + uy * t
    pz = oz[...] + uz * t
    valid = ((t < tmax)
             & (px >= 0.0) & (px < 1.0)
             & (py >= 0.0) & (py < 1.0)
             & (pz >= 0.0) & (pz < 1.0))

    rowi = lax.broadcasted_iota(jnp.int32, (128, CCHUNK), 0)
    coli = lax.broadcasted_iota(jnp.int32, (128, CCHUNK), 1)
    expand = (rowi == coli // 8).astype(jnp.float32)
    densi = lax.dot_general(
        dens[...], expand, (((1,), (0,)), ((), ())),
        preferred_element_type=jnp.float32,
        precision=lax.Precision.HIGHEST)

    mean = gsum[0, 0] / float(NCELL)
    thresh = jnp.minimum(mean, MIN_OPT_THICKNESS)
    occ = valid & (densi > thresh)
    occf = occ.astype(jnp.float32)

    chan = lax.broadcasted_iota(jnp.int32, (RB_C, CCHUNK), 1) % 8
    ones = jnp.ones_like(t)
    res = jnp.where(
        chan == 0, px,
        jnp.where(chan == 1, py,
                  jnp.where(chan == 2, pz,
                            jnp.where(chan == 3, dt,
                                      jnp.where(chan == 4, ux * ones,
                                                jnp.where(chan == 5, uy * ones,
                                                          jnp.where(chan == 6, uz * ones,
                                                                    densi)))))))
    out_ref[...] = res * occf

    @pl.when(c == 0)
    def _():
        nst_ref[...] = jnp.zeros_like(nst_ref)

    nst_ref[...] += jnp.sum(occ.astype(jnp.int32), axis=1, keepdims=True) // 8


def _assemble_call(ray_start, ox, oy, oz, dx, dy, dz, dens, gsum, gi, si,
                   out2_acc=None, nst_acc=None):
    slab = ox.shape[0]
    nb = slab // RB_C
    ncc = (MAX_STEP * 8) // CCHUNK
    sb = ray_start // RB_C
    ray_spec = pl.BlockSpec((RB_C, 1), lambda i, c: (i, 0))
    any_spec = pl.BlockSpec(memory_space=pl.ANY)
    in_specs = [ray_spec] * 6 + [
        pl.BlockSpec((RB_C, CCHUNK // 8), lambda i, c: (i, c)),
        pl.BlockSpec((1, 1), lambda i, c: (0, 0)),
        pl.BlockSpec((1, CCHUNK), lambda i, c: (0, c)),
        pl.BlockSpec((1, CCHUNK), lambda i, c: (0, c)),
    ]
    args = [ox, oy, oz, dx, dy, dz, dens, gsum, gi, si]
    aliases = {}
    if out2_acc is not None:
        in_specs += [any_spec, any_spec]
        args += [out2_acc, nst_acc]
        aliases = {10: 0, 11: 1}
    body = _assemble_body if out2_acc is not None else (
        lambda *a: _assemble_body(*a[:10], None, None, *a[10:]))
    out2, nst = pl.pallas_call(
        body,
        grid=(nb, ncc),
        in_specs=in_specs,
        out_specs=[
            pl.BlockSpec((RB_C, CCHUNK), lambda i, c: (i + sb, c)),
            pl.BlockSpec((RB_C, 1), lambda i, c: (i + sb, 0)),
        ],
        out_shape=[
            jax.ShapeDtypeStruct((N_RAYS, MAX_STEP * 8), jnp.float32),
            jax.ShapeDtypeStruct((N_RAYS, 1), jnp.int32),
        ],
        input_output_aliases=aliases,
    )(*args)
    return out2, nst


def kernel(rays_o, rays_d, density_grid):
    ox, oy, oz = (rays_o[:, i:i + 1] for i in range(3))
    dx, dy, dz = (rays_d[:, i:i + 1] for i in range(3))
    steps = jnp.arange(MAX_STEP, dtype=jnp.float32)
    g = jnp.power(1.0 + CONE, steps).reshape(1, MAX_STEP)
    s = (steps * DT_MIN).reshape(1, MAX_STEP)
    gi = jnp.repeat(g, 8, axis=1)
    si = jnp.repeat(s, 8, axis=1)

    gsum = _gridsum_call(density_grid.reshape(NCELL // MAX_STEP, MAX_STEP))

    starts = [sum(SLABS[:k]) for k in range(len(SLABS))]
    dens_slabs = []
    rays_slabs = []
    for k, (st, sz) in enumerate(zip(starts, SLABS)):
        rs = tuple(a[st:st + sz] for a in (ox, oy, oz, dx, dy, dz))
        flat = _index_call(*rs, g, s)
        dens_slabs.append(_sc_gather_call(flat.reshape(-1), density_grid))
        rays_slabs.append(rs)

    out2, nst = None, None
    for k, (st, sz) in enumerate(zip(starts, SLABS)):
        out2, nst = _assemble_call(
            st, *rays_slabs[k], dens_slabs[k].reshape(sz, MAX_STEP),
            gsum, gi, si, out2, nst)

    coords = out2.reshape(N_RAYS, MAX_STEP, 8)
    return coords, nst.reshape(N_RAYS)
